# baseline (speedup 1.0000x reference)
import functools

import jax
import jax.numpy as jnp
from jax import lax
from jax.experimental import pallas as pl
from jax.experimental.pallas import tpu as pltpu
from jax.experimental.pallas import tpu_sc as plsc

NC, NS, L = 2, 16, 16
NW = NC * NS
B = 128
V = 2
N = B * V
D = 128
RPW = N // NW
NJB = N // L
DC = D // L
INV_T = 20.0
RB = 2

_mesh = plsc.VectorSubcoreMesh(
    core_axis_name="c", subcore_axis_name="s", num_cores=NC, num_subcores=NS
)


@functools.partial(
    pl.kernel,
    out_type=jax.ShapeDtypeStruct((NW, L), jnp.float32),
    mesh=_mesh,
    scratch_types=[
        pltpu.VMEM((D, N), jnp.float32),
        pltpu.VMEM((RPW, D), jnp.float32),
        pltpu.VMEM((B,), jnp.int32),
        pltpu.VMEM((L,), jnp.int32),
        pltpu.VMEM((L,), jnp.float32),
    ],
    compiler_params=pltpu.CompilerParams(needs_layout_passes=False),
)
def _sc_partials(gt_hbm, g_hbm, lbl_hbm, lbl3_hbm, out_hbm,
                 gt_v, gmy_v, lbl_v, lmy_v, part_v):
    wid = lax.axis_index("s") * NC + lax.axis_index("c")
    base = wid * RPW
    pltpu.sync_copy(gt_hbm, gt_v)
    pltpu.sync_copy(g_hbm.at[pl.ds(base, RPW)], gmy_v)
    pltpu.sync_copy(lbl_hbm, lbl_v)
    pltpu.sync_copy(lbl3_hbm.at[pl.ds(base, L)], lmy_v)
    mylab = lmy_v[pl.ds(0, L)]

    total_p = jnp.float32(0.0)
    dsum_p = jnp.float32(0.0)
    cnt_p = jnp.float32(0.0)

    for rb0 in range(0, RPW, RB):
        nrb = min(RB, RPW - rb0)

        def dcbody(dc, accs, rb0=rb0, nrb=nrb):
            accs = list(accs)
            dbase = dc * L
            mych = [gmy_v[rb0 + r, pl.ds(dbase, L)] for r in range(nrb)]
            for dl in range(L):
                ss = [mych[r][dl] for r in range(nrb)]
                for jb in range(NJB):
                    col = gt_v[dbase + dl, pl.ds(jb * L, L)]
                    for r in range(nrb):
                        accs[r * NJB + jb] = accs[r * NJB + jb] + ss[r] * col
            return tuple(accs)

        init = tuple(jnp.zeros((L,), jnp.float32) for _ in range(nrb * NJB))
        accs = lax.fori_loop(0, DC, dcbody, init)

        for r in range(nrb):
            li = mylab[rb0 + r]
            masked = []
            mcount = jnp.zeros((L,), jnp.float32)
            for jb in range(NJB):
                lv = lbl_v[pl.ds((jb % (NJB // V)) * L, L)]
                m = lv != li
                masked.append(jnp.where(m, accs[r * NJB + jb] * INV_T, 0.0))
                mcount = mcount + jnp.where(m, 1.0, 0.0)
            mx = masked[0]
            for jb in range(1, NJB):
                mx = jnp.maximum(mx, masked[jb])
            rowmax = jnp.max(mx)
            es = jnp.zeros((L,), jnp.float32)
            for jb in range(NJB):
                es = es + jnp.exp(masked[jb] - rowmax)
            msum = jnp.sum(mcount)
            total_p = total_p + jnp.sum(es)
            dsum_p = dsum_p + msum
            cnt_p = cnt_p + (msum > 0).astype(jnp.float32)

    lanes = lax.iota(jnp.int32, L)
    partvec = jnp.where(
        lanes == 0, total_p,
        jnp.where(lanes == 1, dsum_p, jnp.where(lanes == 2, cnt_p, 0.0)))
    part_v[...] = partvec
    pltpu.sync_copy(part_v, out_hbm.at[wid])


def _combine_body(p_ref, o_ref):
    p = p_ref[...]
    tot = jnp.sum(p[:, 0:1])
    dsum = jnp.sum(p[:, 1:2])
    cnt = jnp.sum(p[:, 2:3])
    xpos = tot / jnp.maximum(dsum, 1.0)
    n = jnp.float32(N)
    loss = (cnt * jnp.log(xpos + 1e-6) + (n - cnt) * jnp.log(1e-6)) / n
    o_ref[...] = loss.reshape(1, 1)


def kernel(features, labels):
    g = jnp.transpose(features, (1, 0, 2)).reshape(N, D)
    gt = jnp.transpose(features, (2, 1, 0)).reshape(D, N)
    lbl3 = jnp.tile(labels, 3)
    partials = _sc_partials(gt, g, labels, lbl3)
    loss = pl.pallas_call(
        _combine_body,
        out_shape=jax.ShapeDtypeStruct((1, 1), jnp.float32),
    )(partials)
    return loss[0, 0]
